# Initial kernel scaffold; baseline (speedup 1.0000x reference)
import jax
import jax.numpy as jnp
from jax.experimental import pallas as pl


def kernel(user_indices, user_feat, pos_item_indices, pos_item_feat, neg_item_indices, neg_item_feat, comp_neg_indices, emb_user, emb_item, emb_feat_user, emb_feat_item, pop_emb_user, int_emb_user, pop_emb_item, int_emb_item, W_pop, b_pop, W_int, b_int):
    raise NotImplementedError("write your pallas kernel here")


# speedup vs baseline: 4.6582x; 4.6582x over previous
import functools

import jax
import jax.numpy as jnp
from jax import lax
from jax.experimental import pallas as pl
from jax.experimental.pallas import tpu as pltpu
from jax.experimental.pallas import tpu_sc as plsc

B = 16384
D = 32
FL = 26
NBR = 3
NB = NBR * B
NC = 2
NS = 16
NW = NC * NS
SPW = B // NW
CH = 64
NCHUNK = SPW // CH
ROWS_PER_CH = CH * FL
IDXROWS_PER_CH = ROWS_PER_CH // 128


def _sc_gather_body(e_idx, f_idx, emb_user, emb_item, feat_user, feat_item,
                    e_out, fe_out, eidx_v, e_rows, fidx_v, fe_rows, sem):
    wid = lax.axis_index("s") * NC + lax.axis_index("c")
    woff = pl.multiple_of(wid * SPW, SPW)
    for br in range(NBR):
        etab = emb_user if br == 0 else emb_item
        ftab = feat_user if br == 0 else feat_item
        ebase = br * B + woff
        pltpu.sync_copy(e_idx.at[pl.ds(ebase, SPW)], eidx_v)
        cps = [pltpu.async_copy(etab.at[eidx_v.at[pl.ds(j * 128, 128)]],
                                e_rows.at[pl.ds(j * 128, 128)], sem)
               for j in range(SPW // 128)]
        for cp in cps:
            cp.wait()
        pltpu.sync_copy(e_rows, e_out.at[pl.ds(ebase, SPW)])

        fbase = pl.multiple_of(ebase * FL, SPW * FL)
        pltpu.sync_copy(f_idx.at[pl.ds(fbase, SPW * FL)], fidx_v)

        def chunk(c, carry):
            coff = pl.multiple_of(c * ROWS_PER_CH, ROWS_PER_CH)
            cps = [pltpu.async_copy(
                       ftab.at[fidx_v.at[pl.ds(coff + j * 128, 128)]],
                       fe_rows.at[pl.ds(j * 128, 128)], sem)
                   for j in range(IDXROWS_PER_CH)]
            for cp in cps:
                cp.wait()
            pltpu.sync_copy(fe_rows, fe_out.at[pl.ds(fbase + coff, ROWS_PER_CH)])
            return carry

        lax.fori_loop(0, NCHUNK, chunk, 0)


def _sc_gather(e_idx, f_idx, emb_user, emb_item, feat_user, feat_item):
    mesh = plsc.VectorSubcoreMesh(core_axis_name="c", subcore_axis_name="s")
    k = functools.partial(
        pl.kernel, mesh=mesh,
        out_type=[jax.ShapeDtypeStruct((NB, D), jnp.float32),
                  jax.ShapeDtypeStruct((NB * FL, D), jnp.float32)],
        scratch_types=[
            pltpu.VMEM((SPW,), jnp.int32),
            pltpu.VMEM((SPW, D), jnp.float32),
            pltpu.VMEM((SPW * FL,), jnp.int32),
            pltpu.VMEM((ROWS_PER_CH, D), jnp.float32),
            pltpu.SemaphoreType.DMA,
        ],
        compiler_params=pltpu.CompilerParams(use_tc_tiling_on_sc=False),
    )(_sc_gather_body)
    return k(e_idx, f_idx, emb_user, emb_item, feat_user, feat_item)


def _attn_body(vecs_ref, wt_ref, bs_ref, e_ref, fe_ref, feat_ref, out_ref):
    f32 = jnp.float32
    e = e_ref[...]
    fe = fe_ref[...]
    feat = feat_ref[...]
    mask = feat != 0
    jj = lax.broadcasted_iota(jnp.int32, (FL * D, FL), 0)
    ll = lax.broadcasted_iota(jnp.int32, (FL * D, FL), 1)
    et = (jj // D == ll).astype(f32)
    e26 = (lax.broadcasted_iota(jnp.int32, (FL, FL * D), 1) // D
           == lax.broadcasted_iota(jnp.int32, (FL, FL * D), 0)).astype(f32)
    rm = (lax.broadcasted_iota(jnp.int32, (FL * D, D), 0) % D
          == lax.broadcasted_iota(jnp.int32, (FL * D, D), 1)).astype(f32)
    for v in range(2):
        vcol = vecs_ref[0][:, v:v + 1]
        l_e = jnp.dot(e, vcol, preferred_element_type=f32)
        v832 = jnp.dot(rm, vcol, preferred_element_type=f32)
        l_f = jnp.dot(fe, et * v832, preferred_element_type=f32)
        l_f = jnp.where(mask, l_f, -1e30)
        m = jnp.maximum(jnp.max(l_f, axis=1, keepdims=True), l_e)
        w_e = jnp.exp(l_e - m)
        w_f = jnp.exp(l_f - m)
        s = w_e + jnp.sum(w_f, axis=1, keepdims=True)
        w832 = jnp.dot(w_f, e26, preferred_element_type=f32)
        pooled = (e * w_e +
                  jnp.dot(fe * w832, rm, preferred_element_type=f32)) / s
        h = jnp.dot(pooled, wt_ref[v], preferred_element_type=f32)
        out_ref[0, v] = h + bs_ref[v][None, :]


def _attn_call(vecs, wt, bs, e_rows, fe2, feat_all, bb=512, interpret=False):
    nj = B // bb
    return pl.pallas_call(
        _attn_body,
        grid=(NBR, nj),
        in_specs=[
            pl.BlockSpec((1, D, 2), lambda i, j: (i, 0, 0)),
            pl.BlockSpec((2, D, D), lambda i, j: (0, 0, 0)),
            pl.BlockSpec((2, D), lambda i, j: (0, 0)),
            pl.BlockSpec((bb, D), lambda i, j: (i * nj + j, 0)),
            pl.BlockSpec((bb, FL * D), lambda i, j: (i * nj + j, 0)),
            pl.BlockSpec((bb, FL), lambda i, j: (i * nj + j, 0)),
        ],
        out_specs=pl.BlockSpec((1, 2, bb, D), lambda i, j: (i, 0, j, 0)),
        out_shape=jax.ShapeDtypeStruct((NBR, 2, B, D), jnp.float32),
        interpret=interpret,
    )(vecs, wt, bs, e_rows, fe2, feat_all)


def kernel(user_indices, user_feat, pos_item_indices, pos_item_feat,
           neg_item_indices, neg_item_feat, comp_neg_indices,
           emb_user, emb_item, emb_feat_user, emb_feat_item,
           pop_emb_user, int_emb_user, pop_emb_item, int_emb_item,
           W_pop, b_pop, W_int, b_int):
    neg_idx = neg_item_indices[:, 0]
    neg_feat = neg_item_feat[:, 0]
    e_idx = jnp.concatenate([user_indices, pos_item_indices, neg_idx]
                            ).astype(jnp.int32)
    feat_all = jnp.concatenate([user_feat, pos_item_feat, neg_feat], axis=0
                               ).astype(jnp.int32)
    f_idx = feat_all.reshape(NB * FL)

    e_rows, fe_rows = _sc_gather(e_idx, f_idx, emb_user, emb_item,
                                 emb_feat_user, emb_feat_item)
    fe2 = fe_rows.reshape(NB, FL * D)

    vecs = jnp.stack([
        jnp.concatenate([pop_emb_user, int_emb_user], axis=1),
        jnp.concatenate([pop_emb_item, int_emb_item], axis=1),
        jnp.concatenate([pop_emb_item, int_emb_item], axis=1),
    ])
    wt = jnp.stack([W_pop.T, W_int.T])
    bs = jnp.stack([b_pop, b_int])

    out = _attn_call(vecs, wt, bs, e_rows, fe2, feat_all)
    return out.reshape(2 * NBR, B, D)


# speedup vs baseline: 4.8025x; 1.0310x over previous
import functools

import jax
import jax.numpy as jnp
from jax import lax
from jax.experimental import pallas as pl
from jax.experimental.pallas import tpu as pltpu
from jax.experimental.pallas import tpu_sc as plsc

B = 16384
D = 32
FL = 26
NBR = 3
NB = NBR * B
NC = 2
NS = 16
NW = NC * NS
SPW = B // NW
CH = 64
NCHUNK = SPW // CH
ROWS_PER_CH = CH * FL
IDXROWS_PER_CH = ROWS_PER_CH // 128


def _sc_egather_body(e_idx, emb_user, emb_item, e_out, eidx_v, e_rows, sem):
    wid = lax.axis_index("s") * NC + lax.axis_index("c")
    woff = pl.multiple_of(wid * SPW, SPW)
    for br in range(NBR):
        etab = emb_user if br == 0 else emb_item
        ebase = br * B + woff
        pltpu.sync_copy(e_idx.at[pl.ds(ebase, SPW)], eidx_v)
        cps = [pltpu.async_copy(etab.at[eidx_v.at[pl.ds(j * 128, 128)]],
                                e_rows.at[pl.ds(j * 128, 128)], sem)
               for j in range(SPW // 128)]
        for cp in cps:
            cp.wait()
        pltpu.sync_copy(e_rows, e_out.at[pl.ds(ebase, SPW)])


def _sc_fgather_body(f_idx, feat_user, feat_item, fe_out, fidx_v, fe_rows, sem):
    wid = lax.axis_index("s") * NC + lax.axis_index("c")
    woff = pl.multiple_of(wid * SPW, SPW)
    for br in range(NBR):
        ftab = feat_user if br == 0 else feat_item
        fbase = pl.multiple_of((br * B + woff) * FL, SPW * FL)
        pltpu.sync_copy(f_idx.at[pl.ds(fbase, SPW * FL)], fidx_v)

        def chunk(c, carry):
            coff = pl.multiple_of(c * ROWS_PER_CH, ROWS_PER_CH)
            cps = [pltpu.async_copy(
                       ftab.at[fidx_v.at[pl.ds(coff + j * 128, 128)]],
                       fe_rows.at[pl.ds(j * 128, 128)], sem)
                   for j in range(IDXROWS_PER_CH)]
            for cp in cps:
                cp.wait()
            pltpu.sync_copy(fe_rows, fe_out.at[pl.ds(fbase + coff, ROWS_PER_CH)])
            return carry

        lax.fori_loop(0, NCHUNK, chunk, 0)


_SC_PARAMS = pltpu.CompilerParams(use_tc_tiling_on_sc=False)


def _sc_gather(e_idx, f_idx, emb_user, emb_item, feat_user, feat_item):
    mesh = plsc.VectorSubcoreMesh(core_axis_name="c", subcore_axis_name="s")
    ek = functools.partial(
        pl.kernel, mesh=mesh,
        out_type=[jax.ShapeDtypeStruct((NB, D), jnp.float32)],
        scratch_types=[
            pltpu.VMEM((SPW,), jnp.int32),
            pltpu.VMEM((SPW, D), jnp.float32),
            pltpu.SemaphoreType.DMA,
        ],
        compiler_params=_SC_PARAMS,
    )(_sc_egather_body)
    fk = functools.partial(
        pl.kernel, mesh=mesh,
        out_type=[jax.ShapeDtypeStruct((NB * FL, D), jnp.float32)],
        scratch_types=[
            pltpu.VMEM((SPW * FL,), jnp.int32),
            pltpu.VMEM((ROWS_PER_CH, D), jnp.float32),
            pltpu.SemaphoreType.DMA,
        ],
        compiler_params=_SC_PARAMS,
    )(_sc_fgather_body)
    (e_rows,) = ek(e_idx, emb_user, emb_item)
    (fe_rows,) = fk(f_idx, feat_user, feat_item)
    return e_rows, fe_rows


def _attn_body(vecs_ref, wt_ref, bs_ref, e_ref, fe_ref, feat_ref, out_ref):
    f32 = jnp.float32
    bf16 = jnp.bfloat16
    e = e_ref[...]
    fe = fe_ref[...]
    fe16 = fe.astype(bf16)
    feat = feat_ref[...]
    mask = feat != 0
    jj = lax.broadcasted_iota(jnp.int32, (FL * D, FL), 0)
    ll = lax.broadcasted_iota(jnp.int32, (FL * D, FL), 1)
    et = (jj // D == ll).astype(f32)
    e26 = (lax.broadcasted_iota(jnp.int32, (FL, FL * D), 1) // D
           == lax.broadcasted_iota(jnp.int32, (FL, FL * D), 0)).astype(bf16)
    rm = (lax.broadcasted_iota(jnp.int32, (FL * D, D), 0) % D
          == lax.broadcasted_iota(jnp.int32, (FL * D, D), 1)).astype(f32)
    rm16 = rm.astype(bf16)
    for v in range(2):
        vcol = vecs_ref[0][:, v:v + 1]
        l_e = jnp.dot(e, vcol, preferred_element_type=f32)
        v832 = jnp.dot(rm, vcol, preferred_element_type=f32)
        l_f = jnp.dot(fe16, (et * v832).astype(bf16),
                      preferred_element_type=f32)
        l_f = jnp.where(mask, l_f, -1e30)
        m = jnp.maximum(jnp.max(l_f, axis=1, keepdims=True), l_e)
        w_e = jnp.exp(l_e - m)
        w_f = jnp.exp(l_f - m)
        s = w_e + jnp.sum(w_f, axis=1, keepdims=True)
        w832 = jnp.dot(w_f.astype(bf16), e26,
                       preferred_element_type=f32)
        pooled = (e * w_e +
                  jnp.dot((fe * w832).astype(bf16), rm16,
                          preferred_element_type=f32)) / s
        h = jnp.dot(pooled, wt_ref[v], preferred_element_type=f32)
        out_ref[v] = h + bs_ref[v][None, :]


def _attn_call(vecs, wt, bs, e_rows, fe2, feat_all, bb=512, interpret=False):
    nj = B // bb
    return pl.pallas_call(
        _attn_body,
        grid=(NBR, nj),
        in_specs=[
            pl.BlockSpec((1, D, 2), lambda i, j: (i, 0, 0)),
            pl.BlockSpec((2, D, D), lambda i, j: (0, 0, 0)),
            pl.BlockSpec((2, D), lambda i, j: (0, 0)),
            pl.BlockSpec((bb, D), lambda i, j: (i * nj + j, 0)),
            pl.BlockSpec((bb, FL * D), lambda i, j: (i * nj + j, 0)),
            pl.BlockSpec((bb, FL), lambda i, j: (i * nj + j, 0)),
        ],
        out_specs=pl.BlockSpec((2, bb, D), lambda i, j: (i, j, 0)),
        out_shape=jax.ShapeDtypeStruct((2 * NBR, B, D), jnp.float32),
        interpret=interpret,
    )(vecs, wt, bs, e_rows, fe2, feat_all)


def kernel(user_indices, user_feat, pos_item_indices, pos_item_feat,
           neg_item_indices, neg_item_feat, comp_neg_indices,
           emb_user, emb_item, emb_feat_user, emb_feat_item,
           pop_emb_user, int_emb_user, pop_emb_item, int_emb_item,
           W_pop, b_pop, W_int, b_int):
    neg_idx = neg_item_indices[:, 0]
    neg_feat = neg_item_feat[:, 0]
    e_idx = jnp.concatenate([user_indices, pos_item_indices, neg_idx]
                            ).astype(jnp.int32)
    feat_all = jnp.concatenate([user_feat, pos_item_feat, neg_feat], axis=0
                               ).astype(jnp.int32)
    f_idx = feat_all.reshape(NB * FL)

    e_rows, fe_rows = _sc_gather(e_idx, f_idx, emb_user, emb_item,
                                 emb_feat_user, emb_feat_item)
    fe2 = fe_rows.reshape(NB, FL * D)

    vecs = jnp.stack([
        jnp.concatenate([pop_emb_user, int_emb_user], axis=1),
        jnp.concatenate([pop_emb_item, int_emb_item], axis=1),
        jnp.concatenate([pop_emb_item, int_emb_item], axis=1),
    ])
    wt = jnp.stack([W_pop.T, W_int.T])
    bs = jnp.stack([b_pop, b_int])

    return _attn_call(vecs, wt, bs, e_rows, fe2, feat_all)


# speedup vs baseline: 5.4996x; 1.1452x over previous
import functools

import jax
import jax.numpy as jnp
from jax import lax
from jax.experimental import pallas as pl
from jax.experimental.pallas import tpu as pltpu
from jax.experimental.pallas import tpu_sc as plsc

B = 16384
D = 32
FL = 26
NBR = 3
NB = NBR * B
NC = 2
NS = 16
NW = NC * NS
SPW = B // NW
CH = 64
NCHUNK = SPW // CH
ROWS_PER_CH = CH * FL
IDXROWS_PER_CH = ROWS_PER_CH // 128


ECH = 32


def _sc_egather_body(e_idx, emb_user, emb_item, e_out, eidx_v, e8_v,
                     e_rows, sem):
    wid = lax.axis_index("s") * NC + lax.axis_index("c")
    woff = pl.multiple_of(wid * SPW, SPW)
    for br in range(NBR):
        etab = emb_user if br == 0 else emb_item
        ebase = br * B + woff
        pltpu.sync_copy(e_idx.at[pl.ds(ebase, SPW)], eidx_v)

        def chunk(c, carry):
            cbase = pl.multiple_of(c * ECH, ECH)
            idxs = []
            for g in range(ECH // 16):
                vg = eidx_v[pl.ds(cbase + g * 16, 16)]
                idxs.extend(vg[i] for i in range(16))
            cps = []
            for i in range(ECH):
                b8 = pl.multiple_of((idxs[i] // 8) * 8, 8)
                cps.append(pltpu.async_copy(etab.at[pl.ds(b8, 8)],
                                            e8_v.at[i], sem))
            for cp in cps:
                cp.wait()
            for i in range(ECH):
                sub = lax.rem(idxs[i], 8)
                e_rows[cbase + i, pl.ds(0, 16)] = e8_v[i, sub, pl.ds(0, 16)]
                e_rows[cbase + i, pl.ds(16, 16)] = e8_v[i, sub, pl.ds(16, 16)]
            return carry

        lax.fori_loop(0, SPW // ECH, chunk, 0)
        pltpu.sync_copy(e_rows, e_out.at[pl.ds(ebase, SPW)])


def _sc_fgather_body(f_idx, feat_user, feat_item, fe_out, fidx_v, fe_rows, sem):
    wid = lax.axis_index("s") * NC + lax.axis_index("c")
    woff = pl.multiple_of(wid * SPW, SPW)
    for br in range(NBR):
        ftab = feat_user if br == 0 else feat_item
        fbase = pl.multiple_of((br * B + woff) * FL, SPW * FL)
        pltpu.sync_copy(f_idx.at[pl.ds(fbase, SPW * FL)], fidx_v)

        def chunk(c, carry):
            coff = pl.multiple_of(c * ROWS_PER_CH, ROWS_PER_CH)
            cps = [pltpu.async_copy(
                       ftab.at[fidx_v.at[pl.ds(coff + j * 128, 128)]],
                       fe_rows.at[pl.ds(j * 128, 128)], sem)
                   for j in range(IDXROWS_PER_CH)]
            for cp in cps:
                cp.wait()
            pltpu.sync_copy(fe_rows, fe_out.at[pl.ds(fbase + coff, ROWS_PER_CH)])
            return carry

        lax.fori_loop(0, NCHUNK, chunk, 0)


_SC_PARAMS = pltpu.CompilerParams(use_tc_tiling_on_sc=False)


def _sc_gather(e_idx, f_idx, emb_user, emb_item, feat_user, feat_item):
    mesh = plsc.VectorSubcoreMesh(core_axis_name="c", subcore_axis_name="s")
    ek = functools.partial(
        pl.kernel, mesh=mesh,
        out_type=[jax.ShapeDtypeStruct((NB, D), jnp.float32)],
        scratch_types=[
            pltpu.VMEM((SPW,), jnp.int32),
            pltpu.VMEM((ECH, 8, D), jnp.float32),
            pltpu.VMEM((SPW, D), jnp.float32),
            pltpu.SemaphoreType.DMA,
        ],
        compiler_params=pltpu.CompilerParams(use_tc_tiling_on_sc=True),
    )(_sc_egather_body)
    fk = functools.partial(
        pl.kernel, mesh=mesh,
        out_type=[jax.ShapeDtypeStruct((NB * FL, D), jnp.float32)],
        scratch_types=[
            pltpu.VMEM((SPW * FL,), jnp.int32),
            pltpu.VMEM((ROWS_PER_CH, D), jnp.float32),
            pltpu.SemaphoreType.DMA,
        ],
        compiler_params=_SC_PARAMS,
    )(_sc_fgather_body)
    (e_rows,) = ek(e_idx, emb_user, emb_item)
    (fe_rows,) = fk(f_idx, feat_user, feat_item)
    return e_rows, fe_rows


def _attn_body(vecs_ref, wt_ref, bs_ref, e_ref, fe_ref, feat_ref, out_ref):
    f32 = jnp.float32
    bf16 = jnp.bfloat16
    e = e_ref[...]
    fe = fe_ref[...]
    fe16 = fe.astype(bf16)
    feat = feat_ref[...]
    mask = feat != 0
    jj = lax.broadcasted_iota(jnp.int32, (FL * D, FL), 0)
    ll = lax.broadcasted_iota(jnp.int32, (FL * D, FL), 1)
    et = (jj // D == ll).astype(f32)
    e26 = (lax.broadcasted_iota(jnp.int32, (FL, FL * D), 1) // D
           == lax.broadcasted_iota(jnp.int32, (FL, FL * D), 0)).astype(bf16)
    rm = (lax.broadcasted_iota(jnp.int32, (FL * D, D), 0) % D
          == lax.broadcasted_iota(jnp.int32, (FL * D, D), 1)).astype(f32)
    rm16 = rm.astype(bf16)
    for v in range(2):
        vcol = vecs_ref[0][:, v:v + 1]
        l_e = jnp.dot(e, vcol, preferred_element_type=f32)
        v832 = jnp.dot(rm, vcol, preferred_element_type=f32)
        l_f = jnp.dot(fe16, (et * v832).astype(bf16),
                      preferred_element_type=f32)
        l_f = jnp.where(mask, l_f, -1e30)
        m = jnp.maximum(jnp.max(l_f, axis=1, keepdims=True), l_e)
        w_e = jnp.exp(l_e - m)
        w_f = jnp.exp(l_f - m)
        s = w_e + jnp.sum(w_f, axis=1, keepdims=True)
        w832 = jnp.dot(w_f.astype(bf16), e26,
                       preferred_element_type=f32)
        pooled = (e * w_e +
                  jnp.dot((fe * w832).astype(bf16), rm16,
                          preferred_element_type=f32)) / s
        h = jnp.dot(pooled, wt_ref[v], preferred_element_type=f32)
        out_ref[v] = h + bs_ref[v][None, :]


def _attn_call(vecs, wt, bs, e_rows, fe2, feat_all, bb=512, interpret=False):
    nj = B // bb
    return pl.pallas_call(
        _attn_body,
        grid=(NBR, nj),
        in_specs=[
            pl.BlockSpec((1, D, 2), lambda i, j: (i, 0, 0)),
            pl.BlockSpec((2, D, D), lambda i, j: (0, 0, 0)),
            pl.BlockSpec((2, D), lambda i, j: (0, 0)),
            pl.BlockSpec((bb, D), lambda i, j: (i * nj + j, 0)),
            pl.BlockSpec((bb, FL * D), lambda i, j: (i * nj + j, 0)),
            pl.BlockSpec((bb, FL), lambda i, j: (i * nj + j, 0)),
        ],
        out_specs=pl.BlockSpec((2, bb, D), lambda i, j: (i, j, 0)),
        out_shape=jax.ShapeDtypeStruct((2 * NBR, B, D), jnp.float32),
        interpret=interpret,
    )(vecs, wt, bs, e_rows, fe2, feat_all)


def kernel(user_indices, user_feat, pos_item_indices, pos_item_feat,
           neg_item_indices, neg_item_feat, comp_neg_indices,
           emb_user, emb_item, emb_feat_user, emb_feat_item,
           pop_emb_user, int_emb_user, pop_emb_item, int_emb_item,
           W_pop, b_pop, W_int, b_int):
    neg_idx = neg_item_indices[:, 0]
    neg_feat = neg_item_feat[:, 0]
    e_idx = jnp.concatenate([user_indices, pos_item_indices, neg_idx]
                            ).astype(jnp.int32)
    feat_all = jnp.concatenate([user_feat, pos_item_feat, neg_feat], axis=0
                               ).astype(jnp.int32)
    f_idx = feat_all.reshape(NB * FL)

    e_rows, fe_rows = _sc_gather(e_idx, f_idx, emb_user, emb_item,
                                 emb_feat_user, emb_feat_item)
    fe2 = fe_rows.reshape(NB, FL * D)

    vecs = jnp.stack([
        jnp.concatenate([pop_emb_user, int_emb_user], axis=1),
        jnp.concatenate([pop_emb_item, int_emb_item], axis=1),
        jnp.concatenate([pop_emb_item, int_emb_item], axis=1),
    ])
    wt = jnp.stack([W_pop.T, W_int.T])
    bs = jnp.stack([b_pop, b_int])

    return _attn_call(vecs, wt, bs, e_rows, fe2, feat_all)


# speedup vs baseline: 5.5010x; 1.0003x over previous
import functools

import jax
import jax.numpy as jnp
from jax import lax
from jax.experimental import pallas as pl
from jax.experimental.pallas import tpu as pltpu
from jax.experimental.pallas import tpu_sc as plsc

B = 16384
D = 32
FL = 26
NBR = 3
NB = NBR * B
NC = 2
NS = 16
NW = NC * NS
SPW = B // NW
CH = 64
NCHUNK = SPW // CH
ROWS_PER_CH = CH * FL
IDXROWS_PER_CH = ROWS_PER_CH // 128


ECH = 32


def _sc_egather_body(e_idx, emb_user, emb_item, e_out, eidx_v, e8_v,
                     e_rows, sem):
    wid = lax.axis_index("s") * NC + lax.axis_index("c")
    woff = pl.multiple_of(wid * SPW, SPW)
    for br in range(NBR):
        etab = emb_user if br == 0 else emb_item
        ebase = br * B + woff
        pltpu.sync_copy(e_idx.at[pl.ds(ebase, SPW)], eidx_v)

        def chunk(c, carry):
            cbase = pl.multiple_of(c * ECH, ECH)
            idxs = []
            for g in range(ECH // 16):
                vg = eidx_v[pl.ds(cbase + g * 16, 16)]
                idxs.extend(vg[i] for i in range(16))
            cps = []
            for i in range(ECH):
                b8 = pl.multiple_of((idxs[i] // 8) * 8, 8)
                cps.append(pltpu.async_copy(etab.at[pl.ds(b8, 8)],
                                            e8_v.at[i], sem))
            for cp in cps:
                cp.wait()
            for i in range(ECH):
                sub = lax.rem(idxs[i], 8)
                e_rows[cbase + i, pl.ds(0, 16)] = e8_v[i, sub, pl.ds(0, 16)]
                e_rows[cbase + i, pl.ds(16, 16)] = e8_v[i, sub, pl.ds(16, 16)]
            return carry

        lax.fori_loop(0, SPW // ECH, chunk, 0)
        pltpu.sync_copy(e_rows, e_out.at[pl.ds(ebase, SPW)])


def _sc_fgather_body(f_idx, feat_user, feat_item, fe_out, fidx_v, fe_rows, sem):
    wid = lax.axis_index("s") * NC + lax.axis_index("c")
    woff = pl.multiple_of(wid * SPW, SPW)
    for br in range(NBR):
        ftab = feat_user if br == 0 else feat_item
        fbase = pl.multiple_of((br * B + woff) * FL, SPW * FL)
        pltpu.sync_copy(f_idx.at[pl.ds(fbase, SPW * FL)], fidx_v)

        def chunk(c, carry):
            coff = pl.multiple_of(c * ROWS_PER_CH, ROWS_PER_CH)
            cps = [pltpu.async_copy(
                       ftab.at[fidx_v.at[pl.ds(coff + j * 128, 128)]],
                       fe_rows.at[pl.ds(j * 128, 128)], sem)
                   for j in range(IDXROWS_PER_CH)]
            for cp in cps:
                cp.wait()
            pltpu.sync_copy(fe_rows, fe_out.at[pl.ds(fbase + coff, ROWS_PER_CH)])
            return carry

        lax.fori_loop(0, NCHUNK, chunk, 0)


_SC_PARAMS = pltpu.CompilerParams(use_tc_tiling_on_sc=False)


def _sc_gather(e_idx, f_idx, emb_user, emb_item, feat_user, feat_item):
    mesh = plsc.VectorSubcoreMesh(core_axis_name="c", subcore_axis_name="s")
    ek = functools.partial(
        pl.kernel, mesh=mesh,
        out_type=[jax.ShapeDtypeStruct((NB, D), jnp.float32)],
        scratch_types=[
            pltpu.VMEM((SPW,), jnp.int32),
            pltpu.VMEM((ECH, 8, D), jnp.float32),
            pltpu.VMEM((SPW, D), jnp.float32),
            pltpu.SemaphoreType.DMA,
        ],
        compiler_params=pltpu.CompilerParams(use_tc_tiling_on_sc=True),
    )(_sc_egather_body)
    fk = functools.partial(
        pl.kernel, mesh=mesh,
        out_type=[jax.ShapeDtypeStruct((NB * FL, D), jnp.float32)],
        scratch_types=[
            pltpu.VMEM((SPW * FL,), jnp.int32),
            pltpu.VMEM((ROWS_PER_CH, D), jnp.float32),
            pltpu.SemaphoreType.DMA,
        ],
        compiler_params=_SC_PARAMS,
    )(_sc_fgather_body)
    (e_rows,) = ek(e_idx, emb_user, emb_item)
    (fe_rows,) = fk(f_idx, feat_user, feat_item)
    return e_rows, fe_rows


def _attn_body(vecs_ref, wt_ref, bs_ref, e_ref, fe_ref, feat_ref, out_ref):
    f32 = jnp.float32
    bf16 = jnp.bfloat16
    e = e_ref[...]
    bb = e.shape[0]
    fe = fe_ref[...]
    fe16 = fe.astype(bf16)
    feat = feat_ref[...]
    mask = feat != 0
    jj = lax.broadcasted_iota(jnp.int32, (FL * D, FL), 0)
    ll = lax.broadcasted_iota(jnp.int32, (FL * D, FL), 1)
    et = (jj // D == ll).astype(f32)
    e26 = (lax.broadcasted_iota(jnp.int32, (FL, FL * D), 1) // D
           == lax.broadcasted_iota(jnp.int32, (FL, FL * D), 0)).astype(bf16)
    rm = (lax.broadcasted_iota(jnp.int32, (FL * D, D), 0) % D
          == lax.broadcasted_iota(jnp.int32, (FL * D, D), 1)).astype(f32)
    rm16 = rm.astype(bf16)
    for v in range(2):
        vcol = vecs_ref[0][:, v:v + 1]
        l_e = jnp.dot(e, vcol, preferred_element_type=f32)
        v832 = jnp.dot(rm, vcol, preferred_element_type=f32)
        l_f = jnp.dot(fe16, (et * v832).astype(bf16),
                      preferred_element_type=f32)
        l_f = jnp.where(mask, l_f, -1e30)
        m = jnp.maximum(jnp.max(l_f, axis=1, keepdims=True), l_e)
        w_e = jnp.exp(l_e - m)
        w_f = jnp.exp(l_f - m)
        s = w_e + jnp.sum(w_f, axis=1, keepdims=True)
        w832 = jnp.dot(w_f.astype(bf16), e26,
                       preferred_element_type=f32).astype(bf16)
        pooled = (e * w_e +
                  jnp.dot(fe16 * w832, rm16,
                          preferred_element_type=f32)) / s
        h = jnp.dot(pooled, wt_ref[v], preferred_element_type=f32)
        out_ref[v] = h + bs_ref[v][None, :]


def _attn_call(vecs, wt, bs, e_rows, fe1, feat_all, bb=512, interpret=False):
    nj = B // bb
    return pl.pallas_call(
        _attn_body,
        grid=(NBR, nj),
        in_specs=[
            pl.BlockSpec((1, D, 2), lambda i, j: (i, 0, 0)),
            pl.BlockSpec((2, D, D), lambda i, j: (0, 0, 0)),
            pl.BlockSpec((2, D), lambda i, j: (0, 0)),
            pl.BlockSpec((bb, D), lambda i, j: (i * nj + j, 0)),
            pl.BlockSpec((bb, FL * D), lambda i, j: (i * nj + j, 0)),
            pl.BlockSpec((bb, FL), lambda i, j: (i * nj + j, 0)),
        ],
        out_specs=pl.BlockSpec((2, bb, D), lambda i, j: (i, j, 0)),
        out_shape=jax.ShapeDtypeStruct((2 * NBR, B, D), jnp.float32),
        interpret=interpret,
    )(vecs, wt, bs, e_rows, fe1, feat_all)


def kernel(user_indices, user_feat, pos_item_indices, pos_item_feat,
           neg_item_indices, neg_item_feat, comp_neg_indices,
           emb_user, emb_item, emb_feat_user, emb_feat_item,
           pop_emb_user, int_emb_user, pop_emb_item, int_emb_item,
           W_pop, b_pop, W_int, b_int):
    neg_idx = neg_item_indices[:, 0]
    neg_feat = neg_item_feat[:, 0]
    e_idx = jnp.concatenate([user_indices, pos_item_indices, neg_idx]
                            ).astype(jnp.int32)
    feat_all = jnp.concatenate([user_feat, pos_item_feat, neg_feat], axis=0
                               ).astype(jnp.int32)
    f_idx = feat_all.reshape(NB * FL)

    e_rows, fe_rows = _sc_gather(e_idx, f_idx, emb_user, emb_item,
                                 emb_feat_user, emb_feat_item)
    fe1 = fe_rows.reshape(NB, FL * D)

    vecs = jnp.stack([
        jnp.concatenate([pop_emb_user, int_emb_user], axis=1),
        jnp.concatenate([pop_emb_item, int_emb_item], axis=1),
        jnp.concatenate([pop_emb_item, int_emb_item], axis=1),
    ])
    wt = jnp.stack([W_pop.T, W_int.T])
    bs = jnp.stack([b_pop, b_int])

    return _attn_call(vecs, wt, bs, e_rows, fe1, feat_all)


# speedup vs baseline: 10.0628x; 1.8293x over previous
import functools

import jax
import jax.numpy as jnp
from jax import lax
from jax.experimental import pallas as pl
from jax.experimental.pallas import tpu as pltpu
from jax.experimental.pallas import tpu_sc as plsc

B = 16384
D = 32
FL = 26
NBR = 3
NC = 2
NS = 16
NW = NC * NS
SPW = B // NW
CH = 64
NCHUNK = SPW // CH
ROWS_PER_CH = CH * FL
IDXROWS_PER_CH = ROWS_PER_CH // 128


def _sc_fgather_body(f_idx, ftab, fe_out, fidx_v, fe_rows, sem):
    wid = lax.axis_index("s") * NC + lax.axis_index("c")
    fbase = pl.multiple_of(wid * SPW * FL, SPW * FL)
    pltpu.sync_copy(f_idx.at[pl.ds(fbase, SPW * FL)], fidx_v)

    def chunk(c, carry):
        coff = pl.multiple_of(c * ROWS_PER_CH, ROWS_PER_CH)
        cps = [pltpu.async_copy(
                   ftab.at[fidx_v.at[pl.ds(coff + j * 128, 128)]],
                   fe_rows.at[pl.ds(j * 128, 128)], sem)
               for j in range(IDXROWS_PER_CH)]
        for cp in cps:
            cp.wait()
        pltpu.sync_copy(fe_rows, fe_out.at[pl.ds(fbase + coff, ROWS_PER_CH)])
        return carry

    lax.fori_loop(0, NCHUNK, chunk, 0)


def _sc_fgather(f_idx, ftab):
    mesh = plsc.VectorSubcoreMesh(core_axis_name="c", subcore_axis_name="s")
    fk = functools.partial(
        pl.kernel, mesh=mesh,
        out_type=[jax.ShapeDtypeStruct((B * FL, D), jnp.float32)],
        scratch_types=[
            pltpu.VMEM((SPW * FL,), jnp.int32),
            pltpu.VMEM((ROWS_PER_CH, D), jnp.float32),
            pltpu.SemaphoreType.DMA,
        ],
        compiler_params=pltpu.CompilerParams(use_tc_tiling_on_sc=False),
    )(_sc_fgather_body)
    (fe_rows,) = fk(f_idx, ftab)
    return fe_rows


def _attn_body(vecs_ref, wt_ref, bs_ref, e_ref, fe_ref, feat_ref, out_ref):
    f32 = jnp.float32
    bf16 = jnp.bfloat16
    e = e_ref[...]
    fe = fe_ref[...]
    fe16 = fe.astype(bf16)
    feat = feat_ref[...]
    mask = feat != 0
    jj = lax.broadcasted_iota(jnp.int32, (FL * D, FL), 0)
    ll = lax.broadcasted_iota(jnp.int32, (FL * D, FL), 1)
    et = (jj // D == ll).astype(f32)
    e26 = (lax.broadcasted_iota(jnp.int32, (FL, FL * D), 1) // D
           == lax.broadcasted_iota(jnp.int32, (FL, FL * D), 0)).astype(bf16)
    rm = (lax.broadcasted_iota(jnp.int32, (FL * D, D), 0) % D
          == lax.broadcasted_iota(jnp.int32, (FL * D, D), 1)).astype(f32)
    rm16 = rm.astype(bf16)
    for v in range(2):
        vcol = vecs_ref[:, v:v + 1]
        l_e = jnp.dot(e, vcol, preferred_element_type=f32)
        v832 = jnp.dot(rm, vcol, preferred_element_type=f32)
        l_f = jnp.dot(fe16, (et * v832).astype(bf16),
                      preferred_element_type=f32)
        l_f = jnp.where(mask, l_f, -1e30)
        m = jnp.maximum(jnp.max(l_f, axis=1, keepdims=True), l_e)
        w_e = jnp.exp(l_e - m)
        w_f = jnp.exp(l_f - m)
        s = w_e + jnp.sum(w_f, axis=1, keepdims=True)
        w832 = jnp.dot(w_f.astype(bf16), e26,
                       preferred_element_type=f32).astype(bf16)
        pooled = (e * w_e +
                  jnp.dot(fe16 * w832, rm16,
                          preferred_element_type=f32)) / s
        h = jnp.dot(pooled, wt_ref[v], preferred_element_type=f32)
        out_ref[v] = h + bs_ref[v][None, :]


def _attn_call(vecs, wt, bs, e_rows, fe1, feat, bb=2048, interpret=False):
    nj = B // bb
    return pl.pallas_call(
        _attn_body,
        grid=(nj,),
        in_specs=[
            pl.BlockSpec((D, 2), lambda j: (0, 0)),
            pl.BlockSpec((2, D, D), lambda j: (0, 0, 0)),
            pl.BlockSpec((2, D), lambda j: (0, 0)),
            pl.BlockSpec((bb, D), lambda j: (j, 0)),
            pl.BlockSpec((bb, FL * D), lambda j: (j, 0)),
            pl.BlockSpec((bb, FL), lambda j: (j, 0)),
        ],
        out_specs=pl.BlockSpec((2, bb, D), lambda j: (0, j, 0)),
        out_shape=jax.ShapeDtypeStruct((2, B, D), jnp.float32),
        interpret=interpret,
    )(vecs, wt, bs, e_rows, fe1, feat)


def kernel(user_indices, user_feat, pos_item_indices, pos_item_feat,
           neg_item_indices, neg_item_feat, comp_neg_indices,
           emb_user, emb_item, emb_feat_user, emb_feat_item,
           pop_emb_user, int_emb_user, pop_emb_item, int_emb_item,
           W_pop, b_pop, W_int, b_int):
    neg_idx = neg_item_indices[:, 0]
    neg_feat = neg_item_feat[:, 0]
    vecs_u = jnp.concatenate([pop_emb_user, int_emb_user], axis=1)
    vecs_i = jnp.concatenate([pop_emb_item, int_emb_item], axis=1)
    wt = jnp.stack([W_pop.T, W_int.T])
    bs = jnp.stack([b_pop, b_int])

    branches = [
        (user_feat, user_indices, emb_user, emb_feat_user, vecs_u),
        (pos_item_feat, pos_item_indices, emb_item, emb_feat_item, vecs_i),
        (neg_feat, neg_idx, emb_item, emb_feat_item, vecs_i),
    ]
    outs = []
    for feat, idx, etab, ftab, vecs in branches:
        feat = feat.astype(jnp.int32)
        fe_rows = _sc_fgather(feat.reshape(B * FL), ftab)
        e_rows = jnp.take(etab, idx, axis=0)
        outs.append(_attn_call(vecs, wt, bs, e_rows,
                               fe_rows.reshape(B, FL * D), feat))
    return jnp.concatenate(outs, axis=0)
